# baseline (speedup 1.0000x reference)
import dataclasses
import functools

import jax
import jax.numpy as jnp
from jax import lax
from jax.experimental import pallas as pl
from jax.experimental.pallas import tpu as pltpu
from jax.experimental.pallas import tpu_sc as plsc

_NC = 2
_NS = 16
_NW = _NC * _NS
_L = 16
_NBUF = 4
_HP = jax.lax.Precision.HIGHEST

def _ref_dot(a, b):
    return jnp.dot(a.astype(jnp.bfloat16), b.astype(jnp.bfloat16),
                   preferred_element_type=jnp.float32)


def _sc_compiler_params():
    cp = pltpu.CompilerParams()
    if "needs_layout_passes" in pltpu.CompilerParams.__dataclass_fields__:
        cp = dataclasses.replace(cp, needs_layout_passes=False)
    return cp


def _sc_hist(dst2, n):
    epw = dst2.shape[1]
    mesh = plsc.VectorSubcoreMesh(core_axis_name="c", subcore_axis_name="s")

    @functools.partial(
        pl.kernel,
        out_type=jax.ShapeDtypeStruct((_NW, n), jnp.float32),
        mesh=mesh,
        scratch_types=[
            pltpu.VMEM((epw,), jnp.int32),
            pltpu.VMEM((n,), jnp.float32),
        ],
        compiler_params=_sc_compiler_params(),
    )
    def hist_kernel(dst_hbm, hist_hbm, dstv, degv):
        wid = lax.axis_index("s") * _NC + lax.axis_index("c")
        pltpu.sync_copy(dst_hbm.at[wid], dstv)
        zero16 = jnp.zeros((_L,), jnp.float32)

        @pl.loop(0, n // _L)
        def _(i):
            degv[pl.ds(i * _L, _L)] = zero16

        one16 = jnp.ones((_L,), jnp.float32)

        @pl.loop(0, epw // _L)
        def _(i):
            idx = dstv[pl.ds(i * _L, _L)]
            plsc.addupdate_scatter(degv, [idx], one16)

        pltpu.sync_copy(degv, hist_hbm.at[wid])

    return hist_kernel(dst2)


def _sc_scatter(hws, comb, zeros_init):
    n, d = hws.shape
    nw, n_blocks, blk, _, k = comb.shape
    n_pad = zeros_init.shape[0]
    rows_per_tile = n_pad // _NS
    mesh = plsc.VectorSubcoreMesh(core_axis_name="c", subcore_axis_name="s")

    @functools.partial(
        pl.kernel,
        out_type=jax.ShapeDtypeStruct((_NC, n_pad, d), jnp.float32),
        mesh=mesh,
        scratch_types=[
            pltpu.VMEM((blk, 2, k), jnp.int32),
        ] + [pltpu.VMEM((k, d), jnp.float32) for _ in range(_NBUF)]
          + [pltpu.SemaphoreType.DMA for _ in range(_NBUF)]
          + [pltpu.SemaphoreType.DMA,
             pltpu.VMEM_SHARED((n_pad, d), jnp.float32)],
    )
    def scat_kernel(hws_hbm, comb_hbm, zeros_hbm, out_hbm,
                    idxb, *rest):
        rows = rest[:_NBUF]
        gsem = rest[_NBUF:2 * _NBUF]
        ssem = rest[2 * _NBUF]
        acc = rest[2 * _NBUF + 1]
        cid = lax.axis_index("c")
        sid = lax.axis_index("s")
        wid = sid * _NC + cid
        pltpu.sync_copy(zeros_hbm.at[pl.ds(sid * rows_per_tile, rows_per_tile)],
                        acc.at[pl.ds(sid * rows_per_tile, rows_per_tile)])
        plsc.subcore_barrier()

        @pl.loop(0, n_blocks)
        def _(i):
            pltpu.sync_copy(comb_hbm.at[wid, i], idxb)
            gd = [None] * blk
            sd = [None] * blk
            for j in range(min(_NBUF, blk)):
                gd[j] = pltpu.async_copy(
                    hws_hbm.at[idxb.at[j, 0]], rows[j % _NBUF],
                    gsem[j % _NBUF])
            for j in range(blk):
                gd[j].wait()
                sd[j] = pltpu.async_copy(
                    rows[j % _NBUF], acc.at[idxb.at[j, 1]], ssem, add=True)
                if j + _NBUF < blk:
                    sd[j].wait()
                    gd[j + _NBUF] = pltpu.async_copy(
                        hws_hbm.at[idxb.at[j + _NBUF, 0]], rows[j % _NBUF],
                        gsem[j % _NBUF])
            for j in range(max(blk - _NBUF, 0), blk):
                sd[j].wait()

        plsc.subcore_barrier()
        pltpu.sync_copy(
            acc.at[pl.ds(sid * rows_per_tile, rows_per_tile)],
            out_hbm.at[cid].at[pl.ds(sid * rows_per_tile, rows_per_tile)])

    return scat_kernel(hws, comb, zeros_init)


def _dinv_body(hist_ref, dinv_ref):
    hist = hist_ref[...]
    deg = lax.dot_general(hist, jnp.ones((hist.shape[0], 1), jnp.float32),
                          (((0,), (0,)), ((), ())),
                          precision=_HP,
                          preferred_element_type=jnp.float32)
    dinv_ref[...] = lax.rsqrt(1.0 + deg)


def _tc_dinv(hist):
    nw, n = hist.shape
    return pl.pallas_call(
        _dinv_body,
        out_shape=jax.ShapeDtypeStruct((n, 1), jnp.float32),
    )(hist)


def _prep0_body(x_ref, w_ref, dinv_ref, hws_ref):
    xw = _ref_dot(x_ref[...], w_ref[...])
    hws_ref[...] = xw * dinv_ref[...]


def _tc_prep0(x, w0, dinv, bn):
    n, d = x.shape
    nb = n // bn
    return pl.pallas_call(
        _prep0_body,
        grid=(nb,),
        in_specs=[
            pl.BlockSpec((bn, d), lambda i: (i, 0)),
            pl.BlockSpec((d, d), lambda i: (0, 0)),
            pl.BlockSpec((bn, 1), lambda i: (i, 0)),
        ],
        out_specs=pl.BlockSpec((bn, d), lambda i: (i, 0)),
        out_shape=jax.ShapeDtypeStruct((n, d), jnp.float32),
    )(x, w0, dinv)


def _layer_body(acc_ref, hws_ref, dinv_ref, b_ref, w_ref, out_ref):
    dinv = dinv_ref[...]
    h = acc_ref[0] + acc_ref[1] + hws_ref[...]
    h = jnp.maximum(dinv * h + b_ref[...], 0.0)
    out_ref[...] = _ref_dot(h, w_ref[...]) * dinv


def _tc_layer(acc, hws, dinv, b_row, w_next, bn):
    n, d = hws.shape
    nb = n // bn
    return pl.pallas_call(
        _layer_body,
        grid=(nb,),
        in_specs=[
            pl.BlockSpec((_NC, bn, d), lambda i: (0, i, 0)),
            pl.BlockSpec((bn, d), lambda i: (i, 0)),
            pl.BlockSpec((bn, 1), lambda i: (i, 0)),
            pl.BlockSpec((1, d), lambda i: (0, 0)),
            pl.BlockSpec((d, d), lambda i: (0, 0)),
        ],
        out_specs=pl.BlockSpec((bn, d), lambda i: (i, 0)),
        out_shape=jax.ShapeDtypeStruct((n, d), jnp.float32),
    )(acc, hws, dinv, b_row, w_next)


def _final_body(g_graphs, acc_ref, hws_ref, dinv_ref, b_ref, batch_ref,
                mw1_ref, mb1_ref, mw2_ref, mb2_ref, out_ref, s_ref, cnt_ref):
    i = pl.program_id(0)

    @pl.when(i == 0)
    def _():
        s_ref[...] = jnp.zeros_like(s_ref)
        cnt_ref[...] = jnp.zeros_like(cnt_ref)

    dinv = dinv_ref[...]
    h = acc_ref[0] + acc_ref[1] + hws_ref[...]
    h = jnp.maximum(dinv * h + b_ref[...], 0.0)
    bt = batch_ref[0]
    bn = h.shape[0]
    g_iota = lax.broadcasted_iota(jnp.int32, (g_graphs, bn), 0)
    pt = (g_iota == bt).astype(jnp.float32)
    s_ref[...] += jnp.dot(pt, h, precision=_HP,
                          preferred_element_type=jnp.float32)
    cnt_ref[...] += jnp.sum(pt, axis=1, keepdims=True)

    @pl.when(i == pl.num_programs(0) - 1)
    def _():
        g = s_ref[...] / jnp.maximum(cnt_ref[...], 1.0)
        z = jnp.maximum(_ref_dot(g, mw1_ref[...]) + mb1_ref[...], 0.0)
        out_ref[...] = _ref_dot(z, mw2_ref[...]) + mb2_ref[...]


def _tc_final(acc, hws, dinv, b_row, batch3, mw1, mb1_row, mw2, mb2_11, bn, g_graphs):
    n, d = hws.shape
    nb = n // bn
    return pl.pallas_call(
        functools.partial(_final_body, g_graphs),
        grid=(nb,),
        in_specs=[
            pl.BlockSpec((_NC, bn, d), lambda i: (0, i, 0)),
            pl.BlockSpec((bn, d), lambda i: (i, 0)),
            pl.BlockSpec((bn, 1), lambda i: (i, 0)),
            pl.BlockSpec((1, d), lambda i: (0, 0)),
            pl.BlockSpec((1, 1, bn), lambda i: (i, 0, 0)),
            pl.BlockSpec((d, d), lambda i: (0, 0)),
            pl.BlockSpec((1, d), lambda i: (0, 0)),
            pl.BlockSpec((d, 1), lambda i: (0, 0)),
            pl.BlockSpec((1, 1), lambda i: (0, 0)),
        ],
        out_specs=pl.BlockSpec((g_graphs, 1), lambda i: (0, 0)),
        out_shape=jax.ShapeDtypeStruct((g_graphs, 1), jnp.float32),
        scratch_shapes=[
            pltpu.VMEM((g_graphs, d), jnp.float32),
            pltpu.VMEM((g_graphs, 1), jnp.float32),
        ],
    )(acc, hws, dinv, b_row, batch3, mw1, mb1_row, mw2, mb2_11)


def kernel(x, edge_index, edge_attr, batch, W0, b0, W1, b1, W2, b2,
           mW1, mb1, mW2, mb2):
    n, d = x.shape
    e = edge_index.shape[1]
    g_graphs = 64
    bn = 1000
    k = 80
    blk = 8
    n_pad0 = ((n + _NS * 8 - 1) // (_NS * 8)) * (_NS * 8)
    e_pad = ((e + _NW * blk * k - 1) // (_NW * blk * k)) * (_NW * blk * k)
    pad_w = (e_pad - e) // _NW
    d_src = jnp.tile(jax.lax.rem(jnp.arange(pad_w, dtype=jnp.int32) * 41,
                                 jnp.int32(n)).reshape(1, pad_w), (_NW, 1))
    d_dst = jnp.tile(n + jax.lax.rem(jnp.arange(pad_w, dtype=jnp.int32),
                                     jnp.int32(n_pad0 - n)).reshape(1, pad_w),
                     (_NW, 1))
    src_p = jnp.concatenate(
        [edge_index[0].reshape(_NW, e // _NW), d_src], axis=1).reshape(-1)
    dst_p = jnp.concatenate(
        [edge_index[1].reshape(_NW, e // _NW), d_dst], axis=1).reshape(-1)
    epw = e_pad // _NW
    c_chunks = epw // k
    n_blocks = c_chunks // blk

    src3 = src_p.reshape(_NW, c_chunks, k)
    dst3 = dst_p.reshape(_NW, c_chunks, k)
    comb = jnp.stack([src3, dst3], axis=2).reshape(_NW, n_blocks, blk, 2, k)
    dst2 = edge_index[1].reshape(_NW, e // _NW)
    batch3 = batch.reshape(n // bn, 1, bn)

    n_pad = ((n + _NS * 8 - 1) // (_NS * 8)) * (_NS * 8)
    zeros_init = jnp.zeros((n_pad, d), jnp.float32)

    hist = _sc_hist(dst2, n)
    dinv = _tc_dinv(hist)
    hws = _tc_prep0(x, W0, dinv, bn)

    acc = _sc_scatter(hws, comb, zeros_init)
    hws = _tc_layer(acc, hws, dinv, b0.reshape(1, d), W1, bn)
    acc = _sc_scatter(hws, comb, zeros_init)
    hws = _tc_layer(acc, hws, dinv, b1.reshape(1, d), W2, bn)
    acc = _sc_scatter(hws, comb, zeros_init)

    return _tc_final(acc, hws, dinv, b2.reshape(1, d), batch3,
                     mW1, mb1.reshape(1, d), mW2, mb2.reshape(1, 1),
                     bn, g_graphs)


# speedup vs baseline: 1.1237x; 1.1237x over previous
import dataclasses
import functools

import jax
import jax.numpy as jnp
from jax import lax
from jax.experimental import pallas as pl
from jax.experimental.pallas import tpu as pltpu
from jax.experimental.pallas import tpu_sc as plsc

_NC = 2
_NS = 16
_NW = _NC * _NS
_L = 16
_NBUF = 4
_HP = jax.lax.Precision.HIGHEST

def _ref_dot(a, b):
    return jnp.dot(a.astype(jnp.bfloat16), b.astype(jnp.bfloat16),
                   preferred_element_type=jnp.float32)


def _sc_compiler_params():
    cp = pltpu.CompilerParams()
    if "needs_layout_passes" in pltpu.CompilerParams.__dataclass_fields__:
        cp = dataclasses.replace(cp, needs_layout_passes=False)
    return cp


def _sc_hist(dst2, n):
    epw = dst2.shape[1]
    mesh = plsc.VectorSubcoreMesh(core_axis_name="c", subcore_axis_name="s")

    @functools.partial(
        pl.kernel,
        out_type=jax.ShapeDtypeStruct((_NW, n), jnp.float32),
        mesh=mesh,
        scratch_types=[
            pltpu.VMEM((epw,), jnp.int32),
            pltpu.VMEM((n,), jnp.float32),
        ],
        compiler_params=_sc_compiler_params(),
    )
    def hist_kernel(dst_hbm, hist_hbm, dstv, degv):
        wid = lax.axis_index("s") * _NC + lax.axis_index("c")
        pltpu.sync_copy(dst_hbm.at[wid], dstv)
        zero16 = jnp.zeros((_L,), jnp.float32)

        @pl.loop(0, n // _L)
        def _(i):
            degv[pl.ds(i * _L, _L)] = zero16

        one16 = jnp.ones((_L,), jnp.float32)

        @pl.loop(0, epw // _L)
        def _(i):
            idx = dstv[pl.ds(i * _L, _L)]
            plsc.addupdate_scatter(degv, [idx], one16)

        pltpu.sync_copy(degv, hist_hbm.at[wid])

    return hist_kernel(dst2)


def _sc_scatter(hws, comb, zeros_init):
    n, d = hws.shape
    nw, n_blocks, blk, _, k = comb.shape
    n_pad = zeros_init.shape[0]
    rows_per_tile = n_pad // _NS
    mesh = plsc.VectorSubcoreMesh(core_axis_name="c", subcore_axis_name="s")

    @functools.partial(
        pl.kernel,
        out_type=jax.ShapeDtypeStruct((_NC, n_pad, d), jnp.float32),
        mesh=mesh,
        scratch_types=[
            pltpu.VMEM((blk, 2, k), jnp.int32),
        ] + [pltpu.VMEM((k, d), jnp.float32) for _ in range(_NBUF)]
          + [pltpu.SemaphoreType.DMA for _ in range(_NBUF)]
          + [pltpu.SemaphoreType.DMA,
             pltpu.VMEM_SHARED((n_pad, d), jnp.float32)],
    )
    def scat_kernel(hws_hbm, comb_hbm, zeros_hbm, out_hbm,
                    idxb, *rest):
        rows = rest[:_NBUF]
        gsem = rest[_NBUF:2 * _NBUF]
        ssem = rest[2 * _NBUF]
        acc = rest[2 * _NBUF + 1]
        cid = lax.axis_index("c")
        sid = lax.axis_index("s")
        wid = sid * _NC + cid
        pltpu.sync_copy(zeros_hbm.at[pl.ds(sid * rows_per_tile, rows_per_tile)],
                        acc.at[pl.ds(sid * rows_per_tile, rows_per_tile)])
        plsc.subcore_barrier()

        @pl.loop(0, n_blocks)
        def _(i):
            pltpu.sync_copy(comb_hbm.at[wid, i], idxb)
            gd = [None] * blk
            sd = [None] * blk
            for j in range(min(_NBUF, blk)):
                gd[j] = pltpu.async_copy(
                    hws_hbm.at[idxb.at[j, 0]], rows[j % _NBUF],
                    gsem[j % _NBUF])
            for j in range(blk):
                gd[j].wait()
                sd[j] = pltpu.async_copy(
                    rows[j % _NBUF], acc.at[idxb.at[j, 1]], ssem, add=True)
                if j + _NBUF < blk:
                    sd[j].wait()
                    gd[j + _NBUF] = pltpu.async_copy(
                        hws_hbm.at[idxb.at[j + _NBUF, 0]], rows[j % _NBUF],
                        gsem[j % _NBUF])
            for j in range(max(blk - _NBUF, 0), blk):
                sd[j].wait()

        plsc.subcore_barrier()
        pltpu.sync_copy(
            acc.at[pl.ds(sid * rows_per_tile, rows_per_tile)],
            out_hbm.at[cid].at[pl.ds(sid * rows_per_tile, rows_per_tile)])

    return scat_kernel(hws, comb, zeros_init)


def _dinv_body(hist_ref, dinv_ref):
    hist = hist_ref[...]
    deg = lax.dot_general(hist, jnp.ones((hist.shape[0], 1), jnp.float32),
                          (((0,), (0,)), ((), ())),
                          precision=_HP,
                          preferred_element_type=jnp.float32)
    dinv_ref[...] = lax.rsqrt(1.0 + deg)


def _tc_dinv(hist):
    nw, n = hist.shape
    return pl.pallas_call(
        _dinv_body,
        out_shape=jax.ShapeDtypeStruct((n, 1), jnp.float32),
    )(hist)


def _prep0_body(x_ref, w_ref, dinv_ref, hws_ref):
    xw = _ref_dot(x_ref[...], w_ref[...])
    hws_ref[...] = xw * dinv_ref[...]


def _tc_prep0(x, w0, dinv, bn):
    n, d = x.shape
    nb = n // bn
    return pl.pallas_call(
        _prep0_body,
        grid=(nb,),
        in_specs=[
            pl.BlockSpec((bn, d), lambda i: (i, 0)),
            pl.BlockSpec((d, d), lambda i: (0, 0)),
            pl.BlockSpec((bn, 1), lambda i: (i, 0)),
        ],
        out_specs=pl.BlockSpec((bn, d), lambda i: (i, 0)),
        out_shape=jax.ShapeDtypeStruct((n, d), jnp.float32),
    )(x, w0, dinv)


def _layer_body(acc_ref, hws_ref, dinv_ref, b_ref, w_ref, out_ref):
    dinv = dinv_ref[...]
    h = acc_ref[0] + acc_ref[1] + hws_ref[...]
    h = jnp.maximum(dinv * h + b_ref[...], 0.0)
    out_ref[...] = _ref_dot(h, w_ref[...]) * dinv


def _tc_layer(acc, hws, dinv, b_row, w_next, bn):
    n, d = hws.shape
    nb = n // bn
    return pl.pallas_call(
        _layer_body,
        grid=(nb,),
        in_specs=[
            pl.BlockSpec((_NC, bn, d), lambda i: (0, i, 0)),
            pl.BlockSpec((bn, d), lambda i: (i, 0)),
            pl.BlockSpec((bn, 1), lambda i: (i, 0)),
            pl.BlockSpec((1, d), lambda i: (0, 0)),
            pl.BlockSpec((d, d), lambda i: (0, 0)),
        ],
        out_specs=pl.BlockSpec((bn, d), lambda i: (i, 0)),
        out_shape=jax.ShapeDtypeStruct((n, d), jnp.float32),
    )(acc, hws, dinv, b_row, w_next)


def _final_body(g_graphs, acc_ref, hws_ref, dinv_ref, b_ref, batch_ref,
                mw1_ref, mb1_ref, mw2_ref, mb2_ref, out_ref, s_ref, cnt_ref):
    i = pl.program_id(0)

    @pl.when(i == 0)
    def _():
        s_ref[...] = jnp.zeros_like(s_ref)
        cnt_ref[...] = jnp.zeros_like(cnt_ref)

    dinv = dinv_ref[...]
    h = acc_ref[0] + acc_ref[1] + hws_ref[...]
    h = jnp.maximum(dinv * h + b_ref[...], 0.0)
    bt = batch_ref[0]
    bn = h.shape[0]
    g_iota = lax.broadcasted_iota(jnp.int32, (g_graphs, bn), 0)
    pt = (g_iota == bt).astype(jnp.float32)
    s_ref[...] += jnp.dot(pt, h, precision=_HP,
                          preferred_element_type=jnp.float32)
    cnt_ref[...] += jnp.sum(pt, axis=1, keepdims=True)

    @pl.when(i == pl.num_programs(0) - 1)
    def _():
        g = s_ref[...] / jnp.maximum(cnt_ref[...], 1.0)
        z = jnp.maximum(_ref_dot(g, mw1_ref[...]) + mb1_ref[...], 0.0)
        out_ref[...] = _ref_dot(z, mw2_ref[...]) + mb2_ref[...]


def _tc_final(acc, hws, dinv, b_row, batch3, mw1, mb1_row, mw2, mb2_11, bn, g_graphs):
    n, d = hws.shape
    nb = n // bn
    return pl.pallas_call(
        functools.partial(_final_body, g_graphs),
        grid=(nb,),
        in_specs=[
            pl.BlockSpec((_NC, bn, d), lambda i: (0, i, 0)),
            pl.BlockSpec((bn, d), lambda i: (i, 0)),
            pl.BlockSpec((bn, 1), lambda i: (i, 0)),
            pl.BlockSpec((1, d), lambda i: (0, 0)),
            pl.BlockSpec((1, 1, bn), lambda i: (i, 0, 0)),
            pl.BlockSpec((d, d), lambda i: (0, 0)),
            pl.BlockSpec((1, d), lambda i: (0, 0)),
            pl.BlockSpec((d, 1), lambda i: (0, 0)),
            pl.BlockSpec((1, 1), lambda i: (0, 0)),
        ],
        out_specs=pl.BlockSpec((g_graphs, 1), lambda i: (0, 0)),
        out_shape=jax.ShapeDtypeStruct((g_graphs, 1), jnp.float32),
        scratch_shapes=[
            pltpu.VMEM((g_graphs, d), jnp.float32),
            pltpu.VMEM((g_graphs, 1), jnp.float32),
        ],
    )(acc, hws, dinv, b_row, batch3, mw1, mb1_row, mw2, mb2_11)


def kernel(x, edge_index, edge_attr, batch, W0, b0, W1, b1, W2, b2,
           mW1, mb1, mW2, mb2):
    n, d = x.shape
    e = edge_index.shape[1]
    g_graphs = 64
    bn = 1000
    k = 80
    blk = 16
    n_pad0 = ((n + _NS * 8 - 1) // (_NS * 8)) * (_NS * 8)
    e_pad = ((e + _NW * blk * k - 1) // (_NW * blk * k)) * (_NW * blk * k)
    pad_w = (e_pad - e) // _NW
    d_src = jnp.tile(jax.lax.rem(jnp.arange(pad_w, dtype=jnp.int32) * 41,
                                 jnp.int32(n)).reshape(1, pad_w), (_NW, 1))
    d_dst = jnp.tile(n + jax.lax.rem(jnp.arange(pad_w, dtype=jnp.int32),
                                     jnp.int32(n_pad0 - n)).reshape(1, pad_w),
                     (_NW, 1))
    src_p = jnp.concatenate(
        [edge_index[0].reshape(_NW, e // _NW), d_src], axis=1).reshape(-1)
    dst_p = jnp.concatenate(
        [edge_index[1].reshape(_NW, e // _NW), d_dst], axis=1).reshape(-1)
    epw = e_pad // _NW
    c_chunks = epw // k
    n_blocks = c_chunks // blk

    src3 = src_p.reshape(_NW, c_chunks, k)
    dst3 = dst_p.reshape(_NW, c_chunks, k)
    comb = jnp.stack([src3, dst3], axis=2).reshape(_NW, n_blocks, blk, 2, k)
    dst2 = edge_index[1].reshape(_NW, e // _NW)
    batch3 = batch.reshape(n // bn, 1, bn)

    n_pad = ((n + _NS * 8 - 1) // (_NS * 8)) * (_NS * 8)
    zeros_init = jnp.zeros((n_pad, d), jnp.float32)

    hist = _sc_hist(dst2, n)
    dinv = _tc_dinv(hist)
    hws = _tc_prep0(x, W0, dinv, bn)

    acc = _sc_scatter(hws, comb, zeros_init)
    hws = _tc_layer(acc, hws, dinv, b0.reshape(1, d), W1, bn)
    acc = _sc_scatter(hws, comb, zeros_init)
    hws = _tc_layer(acc, hws, dinv, b1.reshape(1, d), W2, bn)
    acc = _sc_scatter(hws, comb, zeros_init)

    return _tc_final(acc, hws, dinv, b2.reshape(1, d), batch3,
                     mW1, mb1.reshape(1, d), mW2, mb2.reshape(1, 1),
                     bn, g_graphs)
